# Initial kernel scaffold; baseline (speedup 1.0000x reference)
import jax
import jax.numpy as jnp
from jax.experimental import pallas as pl


def kernel(states, edges, tw, tb, gk, grk, gb):
    raise NotImplementedError("write your pallas kernel here")


# speedup vs baseline: 13.8336x; 13.8336x over previous
import functools

import jax
import jax.numpy as jnp
from jax import lax
from jax.experimental import pallas as pl
from jax.experimental.pallas import tpu as pltpu
from jax.experimental.pallas import tpu_sc as plsc


def _transform_body(h_ref, tw_ref, tb_ref, out_ref):
    h = h_ref[...]
    T = tw_ref.shape[0]
    for t in range(T):
        out_ref[t] = (
            jnp.dot(h, tw_ref[t], preferred_element_type=jnp.float32)
            + tb_ref[t][None, :]
        )


def _transform(h, tw_l, tb_l, bn):
    N, H = h.shape
    T = tw_l.shape[0]
    nb = N // bn
    return pl.pallas_call(
        _transform_body,
        grid=(nb,),
        in_specs=[
            pl.BlockSpec((bn, H), lambda i: (i, 0)),
            pl.BlockSpec((T, H, H), lambda i: (0, 0, 0)),
            pl.BlockSpec((T, H), lambda i: (0, 0)),
        ],
        out_specs=pl.BlockSpec((T, bn, H), lambda i: (0, i, 0)),
        out_shape=jax.ShapeDtypeStruct((T, N, H), jnp.float32),
    )(h, tw_l, tb_l)


def _gru_body(p_ref, h_ref, gk_ref, grk_ref, gb_ref, out_ref):
    msgs = p_ref[0] + p_ref[1]
    h = h_ref[...]
    H = h.shape[-1]
    xk = jnp.dot(msgs, gk_ref[...], preferred_element_type=jnp.float32) + gb_ref[0][None, :]
    hk = jnp.dot(h, grk_ref[...], preferred_element_type=jnp.float32) + gb_ref[1][None, :]
    z = jax.nn.sigmoid(xk[:, :H] + hk[:, :H])
    r = jax.nn.sigmoid(xk[:, H:2 * H] + hk[:, H:2 * H])
    hh = jnp.tanh(xk[:, 2 * H:] + r * hk[:, 2 * H:])
    out_ref[...] = z * h + (1.0 - z) * hh


def _gru(partials, h, gk_l, grk_l, gb_l, bn):
    N, H = h.shape
    nb = N // bn
    return pl.pallas_call(
        _gru_body,
        grid=(nb,),
        in_specs=[
            pl.BlockSpec((2, bn, H), lambda i: (0, i, 0)),
            pl.BlockSpec((bn, H), lambda i: (i, 0)),
            pl.BlockSpec((H, 3 * H), lambda i: (0, 0)),
            pl.BlockSpec((H, 3 * H), lambda i: (0, 0)),
            pl.BlockSpec((2, 3 * H), lambda i: (0, 0)),
        ],
        out_specs=pl.BlockSpec((bn, H), lambda i: (i, 0)),
        out_shape=jax.ShapeDtypeStruct((N, H), jnp.float32),
    )(partials, h, gk_l, grk_l, gb_l)


@functools.lru_cache(maxsize=None)
def _make_sc_scatter(N, H, E, TN):
    info = plsc.get_sparse_core_info()
    NC, NS = info.num_cores, info.num_subcores
    NW = NC * NS
    assert E % NW == 0
    EW = E // NW
    CH = 128
    nfull = EW // CH
    tail = EW - nfull * CH
    RS8 = (N // NS) // 8 * 8
    rem = N - NS * RS8
    assert rem % 8 == 0 and N % 8 == 0
    mesh = plsc.VectorSubcoreMesh(core_axis_name="c", subcore_axis_name="s")

    scratch = [
        pltpu.VMEM_SHARED((N, H), jnp.float32),
        pltpu.VMEM((CH,), jnp.int32),
        pltpu.VMEM((CH,), jnp.int32),
        pltpu.VMEM((CH, H), jnp.float32),
        pltpu.SemaphoreType.DMA,
    ]
    if tail:
        scratch += [
            pltpu.VMEM((tail,), jnp.int32),
            pltpu.VMEM((tail,), jnp.int32),
            pltpu.VMEM((tail, H), jnp.float32),
        ]

    @functools.partial(
        pl.kernel,
        out_type=jax.ShapeDtypeStruct((NC, N, H), jnp.float32),
        mesh=mesh,
        scratch_types=scratch,
    )
    def sc_kernel(x_hbm, gidx_hbm, dst_hbm, zeros_hbm, out_hbm,
                  acc_sh, gidx_v, dst_v, rows_v, sem, *tail_bufs):
        c = lax.axis_index("c")
        s = lax.axis_index("s")
        w = s * NC + c
        r0 = pl.multiple_of(s * RS8, 8)
        pltpu.sync_copy(zeros_hbm.at[pl.ds(r0, RS8)],
                        acc_sh.at[pl.ds(r0, RS8)])
        if rem:
            @pl.when(s == 0)
            def _zrem():
                pltpu.sync_copy(zeros_hbm.at[pl.ds(NS * RS8, rem)],
                                acc_sh.at[pl.ds(NS * RS8, rem)])
        plsc.subcore_barrier()

        base = w * EW

        @pl.loop(0, nfull)
        def _chunk(j):
            off = pl.multiple_of(base + j * CH, 8)
            pltpu.sync_copy(gidx_hbm.at[pl.ds(off, CH)], gidx_v)
            pltpu.sync_copy(dst_hbm.at[pl.ds(off, CH)], dst_v)
            pltpu.async_copy(x_hbm.at[gidx_v], rows_v, sem).wait()
            pltpu.sync_copy(rows_v, acc_sh.at[dst_v], add=True)

        if tail:
            gidx_t, dst_t, rows_t = tail_bufs
            off = pl.multiple_of(base + nfull * CH, 8)
            pltpu.sync_copy(gidx_hbm.at[pl.ds(off, tail)], gidx_t)
            pltpu.sync_copy(dst_hbm.at[pl.ds(off, tail)], dst_t)
            pltpu.async_copy(x_hbm.at[gidx_t], rows_t, sem).wait()
            pltpu.sync_copy(rows_t, acc_sh.at[dst_t], add=True)

        plsc.subcore_barrier()
        pltpu.sync_copy(acc_sh.at[pl.ds(r0, RS8)],
                        out_hbm.at[c].at[pl.ds(r0, RS8)])
        if rem:
            @pl.when(s == 0)
            def _orem():
                pltpu.sync_copy(acc_sh.at[pl.ds(NS * RS8, rem)],
                                out_hbm.at[c].at[pl.ds(NS * RS8, rem)])

    return sc_kernel


def kernel(states, edges, tw, tb, gk, grk, gb):
    N, H = states.shape
    E = edges.shape[0]
    T = tw.shape[1]
    L = tw.shape[0]
    time_steps = [3, 1]

    types = edges[:, 0]
    src = edges[:, 1]
    dst = edges[:, 2]
    gidx = types * N + src
    zeros = jnp.zeros((N, H), jnp.float32)

    sc_scatter = _make_sc_scatter(N, H, E, T * N)
    bn = 1000

    h = states
    for l, steps in enumerate(time_steps):
        tw_l, tb_l = tw[l], tb[l]
        gk_l, grk_l, gb_l = gk[l], grk[l], gb[l]
        for _ in range(steps):
            x = _transform(h, tw_l, tb_l, bn)
            x = x.reshape(T * N, H)
            partials = sc_scatter(x, gidx, dst, zeros)
            h = _gru(partials, h, gk_l, grk_l, gb_l, bn)
    return h
